# Initial kernel scaffold; baseline (speedup 1.0000x reference)
import jax
import jax.numpy as jnp
from jax.experimental import pallas as pl


def kernel(atoms, atom_positions, graph_indexes, params):
    raise NotImplementedError("write your pallas kernel here")


# speedup vs baseline: 443.4936x; 443.4936x over previous
import jax
import jax.numpy as jnp
from jax import lax
from jax.experimental import pallas as pl
from jax.experimental.pallas import tpu as pltpu

F = 128
RBF = 20
CUT = 5.0
TR = 128
TC = 128
PH = 256


def _embed_kernel(atoms_ref, table_ref, out_ref):
    a = atoms_ref[...]
    cols = lax.broadcasted_iota(jnp.int32, (1, 128), 1)
    onehot = (a == cols).astype(jnp.float32)
    out_ref[...] = jnp.dot(onehot, table_ref[...],
                           preferred_element_type=jnp.float32)


def _phi_kernel(s_ref, w1_ref, b1_ref, w2_ref, b2_ref, out_ref):
    h = jnp.dot(s_ref[...], w1_ref[...],
                preferred_element_type=jnp.float32) + b1_ref[...]
    h = h * jax.nn.sigmoid(h)
    out_ref[...] = jnp.dot(h, w2_ref[...],
                           preferred_element_type=jnp.float32) + b2_ref[...]


def _head_kernel(s_ref, w1_ref, b1_ref, w2_ref, b2_ref, out_ref):
    h = jnp.dot(s_ref[...], w1_ref[...],
                preferred_element_type=jnp.float32) + b1_ref[...]
    h = h * jax.nn.sigmoid(h)
    out_ref[...] = (jnp.sum(h * w2_ref[...], axis=1, keepdims=True)
                    + b2_ref[...])


def _msg_kernel(cinfo_ref, pos_ref, gidx_ref, s_ref, v_ref,
                posT_ref, gidxT_ref, phi_ref,
                wr1_ref, br1_ref, wr2_ref, br2_ref,
                u_ref, vw_ref, w1a_ref, w1b_ref, b1u_ref, w2u_ref, b2u_ref,
                sout_ref, vout_ref):
    t = pl.program_id(0)
    c0 = cinfo_ref[0, t]
    cnt = cinfo_ref[1, t]
    pos_i = pos_ref[...]
    gid_i = gidx_ref[...]
    row_id = t * TR + lax.broadcasted_iota(jnp.int32, (TR, 1), 0)

    def body(c, carry):
        d_si, dvx, dvy, dvz = carry
        cidx = c0 + c
        j0 = cidx * TC
        pj = posT_ref[cidx]
        gj = gidxT_ref[cidx]
        phij = phi_ref[pl.ds(j0, TC), :]
        dx = pj[0:1, :] - pos_i[:, 0:1]
        dy = pj[1:2, :] - pos_i[:, 1:2]
        dz = pj[2:3, :] - pos_i[:, 2:3]
        dist = jnp.sqrt(dx * dx + dy * dy + dz * dz)
        col_id = j0 + lax.broadcasted_iota(jnp.int32, (1, TC), 1)
        m = (gid_i == gj) & (dist < CUT) & (row_id != col_id)
        safe = jnp.where(m, dist, 1.0)
        inv = 1.0 / safe
        x = (jnp.pi / CUT) * safe
        s1 = jnp.sin(x)
        c1 = jnp.cos(x)
        fc = 0.5 * (c1 + 1.0)
        E = jnp.where(m, fc, 0.0)
        w = E * inv
        rhx = dx * inv
        rhy = dy * inv
        rhz = dz * inv
        wx = w * rhx
        wy = w * rhy
        wz = w * rhz
        phi1 = phij[:, :F]
        phi2 = phij[:, F:]
        tt = 2.0 * c1
        sin_p = jnp.zeros_like(s1)
        sin_c = s1
        for r in range(RBF):
            K = w * sin_c
            d_si = d_si + jnp.dot(K, phi2,
                                  preferred_element_type=jnp.float32) * wr2_ref[r:r + 1, :]
            wrow = wr1_ref[r:r + 1, :]
            dvx = dvx + jnp.dot(wx * sin_c, phi1,
                                preferred_element_type=jnp.float32) * wrow
            dvy = dvy + jnp.dot(wy * sin_c, phi1,
                                preferred_element_type=jnp.float32) * wrow
            dvz = dvz + jnp.dot(wz * sin_c, phi1,
                                preferred_element_type=jnp.float32) * wrow
            sin_n = tt * sin_c - sin_p
            sin_p = sin_c
            sin_c = sin_n
        d_si = d_si + jnp.dot(E, phi2,
                              preferred_element_type=jnp.float32) * br2_ref[...]
        br = br1_ref[...]
        dvx = dvx + jnp.dot(E * rhx, phi1,
                            preferred_element_type=jnp.float32) * br
        dvy = dvy + jnp.dot(E * rhy, phi1,
                            preferred_element_type=jnp.float32) * br
        dvz = dvz + jnp.dot(E * rhz, phi1,
                            preferred_element_type=jnp.float32) * br
        return d_si, dvx, dvy, dvz

    zero = jnp.zeros((TR, F), jnp.float32)
    d_si, dvx, dvy, dvz = lax.fori_loop(0, cnt, body, (zero, zero, zero, zero))

    v_in = v_ref[...]
    vnx = v_in[:, 0:F] + dvx
    vny = v_in[:, F:2 * F] + dvy
    vnz = v_in[:, 2 * F:3 * F] + dvz
    U = u_ref[...]
    V = vw_ref[...]
    tUx = jnp.dot(vnx, U, preferred_element_type=jnp.float32)
    tUy = jnp.dot(vny, U, preferred_element_type=jnp.float32)
    tUz = jnp.dot(vnz, U, preferred_element_type=jnp.float32)
    tVx = jnp.dot(vnx, V, preferred_element_type=jnp.float32)
    tVy = jnp.dot(vny, V, preferred_element_type=jnp.float32)
    tVz = jnp.dot(vnz, V, preferred_element_type=jnp.float32)
    vnorm = jnp.sqrt(tVx * tVx + tVy * tVy + tVz * tVz)
    s_in = s_ref[...] + d_si
    h = (jnp.dot(s_in, w1a_ref[...], preferred_element_type=jnp.float32)
         + jnp.dot(vnorm, w1b_ref[...], preferred_element_type=jnp.float32)
         + b1u_ref[...])
    h = h * jax.nn.sigmoid(h)
    a = jnp.dot(h, w2u_ref[...], preferred_element_type=jnp.float32) + b2u_ref[...]
    a_vv = a[:, 0:F]
    a_sv = a[:, F:2 * F]
    a_ss = a[:, 2 * F:3 * F]
    sp = tUx * tVx + tUy * tVy + tUz * tVz
    sout_ref[...] = s_in + sp * a_sv + a_ss
    vout_ref[...] = jnp.concatenate(
        [vnx + tUx * a_vv, vny + tUy * a_vv, vnz + tUz * a_vv], axis=1)


def _full(shape):
    return pl.BlockSpec(shape, lambda t: tuple(0 for _ in shape))


def _row(shape):
    return pl.BlockSpec(shape, lambda t: (t,) + tuple(0 for _ in shape[1:]))


def kernel(atoms, atom_positions, graph_indexes, params):
    N = atoms.shape[0]
    Npad = ((N + PH - 1) // PH) * PH
    pad = Npad - N
    NT = Npad // TR
    NTC = Npad // TC

    atoms_p = jnp.pad(atoms.astype(jnp.int32), (0, pad)).reshape(Npad, 1)
    gmax = graph_indexes[N - 1].astype(jnp.int32)
    gidx_p = jnp.concatenate(
        [graph_indexes.astype(jnp.int32),
         gmax + 1 + jnp.arange(pad, dtype=jnp.int32)])
    pos_p = jnp.pad(atom_positions, ((0, pad), (0, 1)))
    posT3 = pos_p.reshape(NTC, TC, 4).transpose(0, 2, 1)
    gidxT3 = gidx_p.reshape(NTC, 1, TC)
    gidx_col = gidx_p.reshape(Npad, 1)

    gr = gidx_p.reshape(NT, TR)
    lo = jnp.searchsorted(gidx_p, gr[:, 0], side='left').astype(jnp.int32)
    hi = jnp.searchsorted(gidx_p, gr[:, -1], side='right').astype(jnp.int32)
    c0 = lo // TC
    cnt = (hi - 1) // TC - c0 + 1
    cinfo = jnp.stack([c0, cnt]).astype(jnp.int32)

    table = jnp.pad(params['embedding'], ((0, 128 - params['embedding'].shape[0]), (0, 0)))

    s0 = pl.pallas_call(
        _embed_kernel,
        grid=(Npad // PH,),
        in_specs=[_row((PH, 1)), _full((128, F))],
        out_specs=_row((PH, F)),
        out_shape=jax.ShapeDtypeStruct((Npad, F), jnp.float32),
    )(atoms_p, table)
    s = s0
    v = jnp.zeros((Npad, 3 * F), jnp.float32)

    for l in range(3):
        mp = params['msg'][l]
        up = params['upd'][l]
        phi = pl.pallas_call(
            _phi_kernel,
            grid=(Npad // PH,),
            in_specs=[_row((PH, F)), _full((F, F)), _full((1, F)),
                      _full((F, 2 * F)), _full((1, 2 * F))],
            out_specs=_row((PH, 2 * F)),
            out_shape=jax.ShapeDtypeStruct((Npad, 2 * F), jnp.float32),
        )(s0, mp['sj1']['W'], mp['sj1']['b'].reshape(1, F),
          mp['sj2']['W'][:, F:], mp['sj2']['b'][F:].reshape(1, 2 * F))

        wr1 = jnp.pad(mp['rbf']['W'][:, F:2 * F], ((0, 32 - RBF), (0, 0)))
        wr2 = jnp.pad(mp['rbf']['W'][:, 2 * F:], ((0, 32 - RBF), (0, 0)))
        br1 = mp['rbf']['b'][F:2 * F].reshape(1, F)
        br2 = mp['rbf']['b'][2 * F:].reshape(1, F)

        s, v = pl.pallas_call(
            _msg_kernel,
            grid=(NT,),
            in_specs=[
                pl.BlockSpec(memory_space=pltpu.SMEM),
                _row((TR, 4)), _row((TR, 1)), _row((TR, F)), _row((TR, 3 * F)),
                _full((NTC, 4, TC)), _full((NTC, 1, TC)), _full((Npad, 2 * F)),
                _full((32, F)), _full((1, F)), _full((32, F)), _full((1, F)),
                _full((F, F)), _full((F, F)), _full((F, F)), _full((F, F)),
                _full((1, F)), _full((F, 3 * F)), _full((1, 3 * F)),
            ],
            out_specs=[_row((TR, F)), _row((TR, 3 * F))],
            out_shape=[jax.ShapeDtypeStruct((Npad, F), jnp.float32),
                       jax.ShapeDtypeStruct((Npad, 3 * F), jnp.float32)],
        )(cinfo, pos_p, gidx_col, s, v, posT3, gidxT3, phi,
          wr1, br1, wr2, br2,
          up['U']['W'], up['V']['W'],
          up['sj1']['W'][:F], up['sj1']['W'][F:], up['sj1']['b'].reshape(1, F),
          up['sj2']['W'], up['sj2']['b'].reshape(1, 3 * F))

    out = pl.pallas_call(
        _head_kernel,
        grid=(Npad // PH,),
        in_specs=[_row((PH, F)), _full((F, F // 2)), _full((1, F // 2)),
                  _full((1, F // 2)), _full((1, 1))],
        out_specs=_row((PH, 1)),
        out_shape=jax.ShapeDtypeStruct((Npad, 1), jnp.float32),
    )(s, params['f1']['W'], params['f1']['b'].reshape(1, F // 2),
      params['f2']['W'].reshape(1, F // 2), params['f2']['b'].reshape(1, 1))

    return out[:N]


# speedup vs baseline: 444.0093x; 1.0012x over previous
import jax
import jax.numpy as jnp
from jax import lax
from jax.experimental import pallas as pl
from jax.experimental.pallas import tpu as pltpu

F = 128
RBF = 20
CUT = 5.0
TR = 128
TC = 128
PH = 256


def _embed_kernel(atoms_ref, table_ref, out_ref):
    a = atoms_ref[...]
    cols = lax.broadcasted_iota(jnp.int32, (1, 128), 1)
    onehot = (a == cols).astype(jnp.float32)
    out_ref[...] = jnp.dot(onehot, table_ref[...],
                           preferred_element_type=jnp.float32)


def _phi_kernel(s_ref, w1_ref, b1_ref, w2_ref, b2_ref, out_ref):
    h = jnp.dot(s_ref[...], w1_ref[...],
                preferred_element_type=jnp.float32) + b1_ref[...]
    h = h * jax.nn.sigmoid(h)
    out_ref[...] = jnp.dot(h, w2_ref[...],
                           preferred_element_type=jnp.float32) + b2_ref[...]


def _head_kernel(s_ref, w1_ref, b1_ref, w2_ref, b2_ref, out_ref):
    h = jnp.dot(s_ref[...], w1_ref[...],
                preferred_element_type=jnp.float32) + b1_ref[...]
    h = h * jax.nn.sigmoid(h)
    out_ref[...] = (jnp.sum(h * w2_ref[...], axis=1, keepdims=True)
                    + b2_ref[...])


def _msg_kernel(cinfo_ref, pos_ref, gidx_ref, s_ref, v_ref,
                posT_ref, gidxT_ref, phi_ref,
                wr1_ref, br1_ref, wr2_ref, br2_ref,
                u_ref, vw_ref, w1a_ref, w1b_ref, b1u_ref, w2u_ref, b2u_ref,
                sout_ref, vout_ref):
    t = pl.program_id(0)
    c0 = cinfo_ref[0, t]
    cnt = cinfo_ref[1, t]
    pos_i = pos_ref[...]
    gid_i = gidx_ref[...]
    row_id = t * TR + lax.broadcasted_iota(jnp.int32, (TR, 1), 0)

    def body(c, carry):
        d_si, dvx, dvy, dvz = carry
        cidx = c0 + c
        j0 = cidx * TC
        pj = posT_ref[cidx]
        gj = gidxT_ref[cidx]
        phij = phi_ref[pl.ds(j0, TC), :]
        dx = pj[0:1, :] - pos_i[:, 0:1]
        dy = pj[1:2, :] - pos_i[:, 1:2]
        dz = pj[2:3, :] - pos_i[:, 2:3]
        dist = jnp.sqrt(dx * dx + dy * dy + dz * dz)
        col_id = j0 + lax.broadcasted_iota(jnp.int32, (1, TC), 1)
        m = (gid_i == gj) & (dist < CUT) & (row_id != col_id)
        safe = jnp.where(m, dist, 1.0)
        inv = 1.0 / safe
        x = (jnp.pi / CUT) * safe
        s1 = jnp.sin(x)
        c1 = jnp.cos(x)
        fc = 0.5 * (c1 + 1.0)
        E = jnp.where(m, fc, 0.0)
        w = E * inv
        rhx = dx * inv
        rhy = dy * inv
        rhz = dz * inv
        wx = w * rhx
        wy = w * rhy
        wz = w * rhz
        phi1 = phij[:, :F].astype(jnp.bfloat16)
        phi2 = phij[:, F:].astype(jnp.bfloat16)
        tt = 2.0 * c1
        sin_p = jnp.zeros_like(s1)
        sin_c = s1
        for r in range(RBF):
            K = (w * sin_c).astype(jnp.bfloat16)
            d_si = d_si + jnp.dot(K, phi2,
                                  preferred_element_type=jnp.float32) * wr2_ref[r:r + 1, :]
            wrow = wr1_ref[r:r + 1, :]
            dvx = dvx + jnp.dot((wx * sin_c).astype(jnp.bfloat16), phi1,
                                preferred_element_type=jnp.float32) * wrow
            dvy = dvy + jnp.dot((wy * sin_c).astype(jnp.bfloat16), phi1,
                                preferred_element_type=jnp.float32) * wrow
            dvz = dvz + jnp.dot((wz * sin_c).astype(jnp.bfloat16), phi1,
                                preferred_element_type=jnp.float32) * wrow
            sin_n = tt * sin_c - sin_p
            sin_p = sin_c
            sin_c = sin_n
        d_si = d_si + jnp.dot(E.astype(jnp.bfloat16), phi2,
                              preferred_element_type=jnp.float32) * br2_ref[...]
        br = br1_ref[...]
        dvx = dvx + jnp.dot((E * rhx).astype(jnp.bfloat16), phi1,
                            preferred_element_type=jnp.float32) * br
        dvy = dvy + jnp.dot((E * rhy).astype(jnp.bfloat16), phi1,
                            preferred_element_type=jnp.float32) * br
        dvz = dvz + jnp.dot((E * rhz).astype(jnp.bfloat16), phi1,
                            preferred_element_type=jnp.float32) * br
        return d_si, dvx, dvy, dvz

    zero = jnp.zeros((TR, F), jnp.float32)
    d_si, dvx, dvy, dvz = lax.fori_loop(0, cnt, body, (zero, zero, zero, zero))

    v_in = v_ref[...]
    vnx = v_in[:, 0:F] + dvx
    vny = v_in[:, F:2 * F] + dvy
    vnz = v_in[:, 2 * F:3 * F] + dvz
    U = u_ref[...]
    V = vw_ref[...]
    tUx = jnp.dot(vnx, U, preferred_element_type=jnp.float32)
    tUy = jnp.dot(vny, U, preferred_element_type=jnp.float32)
    tUz = jnp.dot(vnz, U, preferred_element_type=jnp.float32)
    tVx = jnp.dot(vnx, V, preferred_element_type=jnp.float32)
    tVy = jnp.dot(vny, V, preferred_element_type=jnp.float32)
    tVz = jnp.dot(vnz, V, preferred_element_type=jnp.float32)
    vnorm = jnp.sqrt(tVx * tVx + tVy * tVy + tVz * tVz)
    s_in = s_ref[...] + d_si
    h = (jnp.dot(s_in, w1a_ref[...], preferred_element_type=jnp.float32)
         + jnp.dot(vnorm, w1b_ref[...], preferred_element_type=jnp.float32)
         + b1u_ref[...])
    h = h * jax.nn.sigmoid(h)
    a = jnp.dot(h, w2u_ref[...], preferred_element_type=jnp.float32) + b2u_ref[...]
    a_vv = a[:, 0:F]
    a_sv = a[:, F:2 * F]
    a_ss = a[:, 2 * F:3 * F]
    sp = tUx * tVx + tUy * tVy + tUz * tVz
    sout_ref[...] = s_in + sp * a_sv + a_ss
    vout_ref[...] = jnp.concatenate(
        [vnx + tUx * a_vv, vny + tUy * a_vv, vnz + tUz * a_vv], axis=1)


def _full(shape):
    return pl.BlockSpec(shape, lambda t: tuple(0 for _ in shape))


def _row(shape):
    return pl.BlockSpec(shape, lambda t: (t,) + tuple(0 for _ in shape[1:]))


def kernel(atoms, atom_positions, graph_indexes, params):
    N = atoms.shape[0]
    Npad = ((N + PH - 1) // PH) * PH
    pad = Npad - N
    NT = Npad // TR
    NTC = Npad // TC

    atoms_p = jnp.pad(atoms.astype(jnp.int32), (0, pad)).reshape(Npad, 1)
    gmax = graph_indexes[N - 1].astype(jnp.int32)
    gidx_p = jnp.concatenate(
        [graph_indexes.astype(jnp.int32),
         gmax + 1 + jnp.arange(pad, dtype=jnp.int32)])
    pos_p = jnp.pad(atom_positions, ((0, pad), (0, 1)))
    posT3 = pos_p.reshape(NTC, TC, 4).transpose(0, 2, 1)
    gidxT3 = gidx_p.reshape(NTC, 1, TC)
    gidx_col = gidx_p.reshape(Npad, 1)

    gr = gidx_p.reshape(NT, TR)
    lo = jnp.searchsorted(gidx_p, gr[:, 0], side='left').astype(jnp.int32)
    hi = jnp.searchsorted(gidx_p, gr[:, -1], side='right').astype(jnp.int32)
    c0 = lo // TC
    cnt = (hi - 1) // TC - c0 + 1
    cinfo = jnp.stack([c0, cnt]).astype(jnp.int32)

    table = jnp.pad(params['embedding'], ((0, 128 - params['embedding'].shape[0]), (0, 0)))

    s0 = pl.pallas_call(
        _embed_kernel,
        grid=(Npad // PH,),
        in_specs=[_row((PH, 1)), _full((128, F))],
        out_specs=_row((PH, F)),
        out_shape=jax.ShapeDtypeStruct((Npad, F), jnp.float32),
    )(atoms_p, table)
    s = s0
    v = jnp.zeros((Npad, 3 * F), jnp.float32)

    for l in range(3):
        mp = params['msg'][l]
        up = params['upd'][l]
        phi = pl.pallas_call(
            _phi_kernel,
            grid=(Npad // PH,),
            in_specs=[_row((PH, F)), _full((F, F)), _full((1, F)),
                      _full((F, 2 * F)), _full((1, 2 * F))],
            out_specs=_row((PH, 2 * F)),
            out_shape=jax.ShapeDtypeStruct((Npad, 2 * F), jnp.float32),
        )(s0, mp['sj1']['W'], mp['sj1']['b'].reshape(1, F),
          mp['sj2']['W'][:, F:], mp['sj2']['b'][F:].reshape(1, 2 * F))

        wr1 = jnp.pad(mp['rbf']['W'][:, F:2 * F], ((0, 32 - RBF), (0, 0)))
        wr2 = jnp.pad(mp['rbf']['W'][:, 2 * F:], ((0, 32 - RBF), (0, 0)))
        br1 = mp['rbf']['b'][F:2 * F].reshape(1, F)
        br2 = mp['rbf']['b'][2 * F:].reshape(1, F)

        s, v = pl.pallas_call(
            _msg_kernel,
            grid=(NT,),
            in_specs=[
                pl.BlockSpec(memory_space=pltpu.SMEM),
                _row((TR, 4)), _row((TR, 1)), _row((TR, F)), _row((TR, 3 * F)),
                _full((NTC, 4, TC)), _full((NTC, 1, TC)), _full((Npad, 2 * F)),
                _full((32, F)), _full((1, F)), _full((32, F)), _full((1, F)),
                _full((F, F)), _full((F, F)), _full((F, F)), _full((F, F)),
                _full((1, F)), _full((F, 3 * F)), _full((1, 3 * F)),
            ],
            out_specs=[_row((TR, F)), _row((TR, 3 * F))],
            out_shape=[jax.ShapeDtypeStruct((Npad, F), jnp.float32),
                       jax.ShapeDtypeStruct((Npad, 3 * F), jnp.float32)],
        )(cinfo, pos_p, gidx_col, s, v, posT3, gidxT3, phi,
          wr1, br1, wr2, br2,
          up['U']['W'], up['V']['W'],
          up['sj1']['W'][:F], up['sj1']['W'][F:], up['sj1']['b'].reshape(1, F),
          up['sj2']['W'], up['sj2']['b'].reshape(1, 3 * F))

    out = pl.pallas_call(
        _head_kernel,
        grid=(Npad // PH,),
        in_specs=[_row((PH, F)), _full((F, F // 2)), _full((1, F // 2)),
                  _full((1, F // 2)), _full((1, 1))],
        out_specs=_row((PH, 1)),
        out_shape=jax.ShapeDtypeStruct((Npad, 1), jnp.float32),
    )(s, params['f1']['W'], params['f1']['b'].reshape(1, F // 2),
      params['f2']['W'].reshape(1, F // 2), params['f2']['b'].reshape(1, 1))

    return out[:N]


# speedup vs baseline: 477.6373x; 1.0757x over previous
import jax
import jax.numpy as jnp
from jax import lax
from jax.experimental import pallas as pl
from jax.experimental.pallas import tpu as pltpu

F = 128
RBF = 20
CUT = 5.0
TR = 128
TC = 128
PH = 256


def _embed_kernel(atoms_ref, table_ref, out_ref):
    a = atoms_ref[...]
    cols = lax.broadcasted_iota(jnp.int32, (1, 128), 1)
    onehot = (a == cols).astype(jnp.float32)
    out_ref[...] = jnp.dot(onehot, table_ref[...],
                           preferred_element_type=jnp.float32)


def _phi_kernel(s_ref, w1_ref, b1_ref, w2_ref, b2_ref, out_ref):
    h = jnp.dot(s_ref[...], w1_ref[...],
                preferred_element_type=jnp.float32) + b1_ref[...]
    h = h * jax.nn.sigmoid(h)
    out_ref[...] = jnp.dot(h, w2_ref[...],
                           preferred_element_type=jnp.float32) + b2_ref[...]


def _head_kernel(s_ref, w1_ref, b1_ref, w2_ref, b2_ref, out_ref):
    h = jnp.dot(s_ref[...], w1_ref[...],
                preferred_element_type=jnp.float32) + b1_ref[...]
    h = h * jax.nn.sigmoid(h)
    out_ref[...] = (jnp.sum(h * w2_ref[...], axis=1, keepdims=True)
                    + b2_ref[...])


def _msg_kernel(cinfo_ref, pos_ref, gidx_ref, s_ref, v_ref,
                posT_ref, gidxT_ref, phi_ref,
                w1r_ref, w2r_ref,
                u_ref, vw_ref, w1a_ref, w1b_ref, b1u_ref, w2u_ref, b2u_ref,
                sout_ref, vout_ref):
    t = pl.program_id(0)
    c0 = cinfo_ref[0, t]
    cnt = cinfo_ref[1, t]
    pos_i = pos_ref[...]
    gid_i = gidx_ref[...]
    row_id = t * TR + lax.broadcasted_iota(jnp.int32, (TR, 1), 0)

    def body(c, carry):
        d_si, dvx, dvy, dvz = carry
        cidx = c0 + c
        j0 = cidx * TC
        pj = posT_ref[cidx]
        gj = gidxT_ref[cidx]
        phij = phi_ref[pl.ds(j0, TC), :]
        dx = pj[0:1, :] - pos_i[:, 0:1]
        dy = pj[1:2, :] - pos_i[:, 1:2]
        dz = pj[2:3, :] - pos_i[:, 2:3]
        dist = jnp.sqrt(dx * dx + dy * dy + dz * dz)
        col_id = j0 + lax.broadcasted_iota(jnp.int32, (1, TC), 1)
        m = (gid_i == gj) & (dist < CUT) & (row_id != col_id)
        safe = jnp.where(m, dist, 1.0)
        inv = 1.0 / safe
        x = (jnp.pi / CUT) * safe
        s1 = jnp.sin(x)
        c1 = jnp.cos(x)
        fc = 0.5 * (c1 + 1.0)
        E = jnp.where(m, fc, 0.0)
        w = E * inv
        rhx = dx * inv
        rhy = dy * inv
        rhz = dz * inv
        wx = w * rhx
        wy = w * rhy
        wz = w * rhz
        phi1 = phij[:, :F]
        phi2 = phij[:, F:]
        tt = 2.0 * c1
        sin_p = jnp.zeros_like(s1)
        sin_c = s1
        ks, kx, ky, kz, r2, r1 = [], [], [], [], [], []
        for r in range(RBF + 1):
            basis = sin_c if r < RBF else safe
            ks.append((w * basis).astype(jnp.bfloat16))
            kx.append((wx * basis).astype(jnp.bfloat16))
            ky.append((wy * basis).astype(jnp.bfloat16))
            kz.append((wz * basis).astype(jnp.bfloat16))
            r2.append((phi2 * w2r_ref[r:r + 1, :]).astype(jnp.bfloat16))
            r1.append((phi1 * w1r_ref[r:r + 1, :]).astype(jnp.bfloat16))
            if r < RBF:
                sin_n = tt * sin_c - sin_p
                sin_p = sin_c
                sin_c = sin_n
        Ks = jnp.concatenate(ks, axis=1)
        Kx = jnp.concatenate(kx, axis=1)
        Ky = jnp.concatenate(ky, axis=1)
        Kz = jnp.concatenate(kz, axis=1)
        R2 = jnp.concatenate(r2, axis=0)
        R1 = jnp.concatenate(r1, axis=0)
        d_si = d_si + jnp.dot(Ks, R2, preferred_element_type=jnp.float32)
        dvx = dvx + jnp.dot(Kx, R1, preferred_element_type=jnp.float32)
        dvy = dvy + jnp.dot(Ky, R1, preferred_element_type=jnp.float32)
        dvz = dvz + jnp.dot(Kz, R1, preferred_element_type=jnp.float32)
        return d_si, dvx, dvy, dvz

    zero = jnp.zeros((TR, F), jnp.float32)
    d_si, dvx, dvy, dvz = lax.fori_loop(0, cnt, body, (zero, zero, zero, zero))

    v_in = v_ref[...]
    vnx = v_in[:, 0:F] + dvx
    vny = v_in[:, F:2 * F] + dvy
    vnz = v_in[:, 2 * F:3 * F] + dvz
    U = u_ref[...]
    V = vw_ref[...]
    tUx = jnp.dot(vnx, U, preferred_element_type=jnp.float32)
    tUy = jnp.dot(vny, U, preferred_element_type=jnp.float32)
    tUz = jnp.dot(vnz, U, preferred_element_type=jnp.float32)
    tVx = jnp.dot(vnx, V, preferred_element_type=jnp.float32)
    tVy = jnp.dot(vny, V, preferred_element_type=jnp.float32)
    tVz = jnp.dot(vnz, V, preferred_element_type=jnp.float32)
    vnorm = jnp.sqrt(tVx * tVx + tVy * tVy + tVz * tVz)
    s_in = s_ref[...] + d_si
    h = (jnp.dot(s_in, w1a_ref[...], preferred_element_type=jnp.float32)
         + jnp.dot(vnorm, w1b_ref[...], preferred_element_type=jnp.float32)
         + b1u_ref[...])
    h = h * jax.nn.sigmoid(h)
    a = jnp.dot(h, w2u_ref[...], preferred_element_type=jnp.float32) + b2u_ref[...]
    a_vv = a[:, 0:F]
    a_sv = a[:, F:2 * F]
    a_ss = a[:, 2 * F:3 * F]
    sp = tUx * tVx + tUy * tVy + tUz * tVz
    sout_ref[...] = s_in + sp * a_sv + a_ss
    vout_ref[...] = jnp.concatenate(
        [vnx + tUx * a_vv, vny + tUy * a_vv, vnz + tUz * a_vv], axis=1)


def _full(shape):
    return pl.BlockSpec(shape, lambda t: tuple(0 for _ in shape))


def _row(shape):
    return pl.BlockSpec(shape, lambda t: (t,) + tuple(0 for _ in shape[1:]))


def kernel(atoms, atom_positions, graph_indexes, params):
    N = atoms.shape[0]
    Npad = ((N + PH - 1) // PH) * PH
    pad = Npad - N
    NT = Npad // TR
    NTC = Npad // TC

    atoms_p = jnp.pad(atoms.astype(jnp.int32), (0, pad)).reshape(Npad, 1)
    gmax = graph_indexes[N - 1].astype(jnp.int32)
    gidx_p = jnp.concatenate(
        [graph_indexes.astype(jnp.int32),
         gmax + 1 + jnp.arange(pad, dtype=jnp.int32)])
    pos_p = jnp.pad(atom_positions, ((0, pad), (0, 1)))
    posT3 = pos_p.reshape(NTC, TC, 4).transpose(0, 2, 1)
    gidxT3 = gidx_p.reshape(NTC, 1, TC)
    gidx_col = gidx_p.reshape(Npad, 1)

    gr = gidx_p.reshape(NT, TR)
    lo = jnp.searchsorted(gidx_p, gr[:, 0], side='left').astype(jnp.int32)
    hi = jnp.searchsorted(gidx_p, gr[:, -1], side='right').astype(jnp.int32)
    c0 = lo // TC
    cnt = (hi - 1) // TC - c0 + 1
    cinfo = jnp.stack([c0, cnt]).astype(jnp.int32)

    table = jnp.pad(params['embedding'], ((0, 128 - params['embedding'].shape[0]), (0, 0)))

    s0 = pl.pallas_call(
        _embed_kernel,
        grid=(Npad // PH,),
        in_specs=[_row((PH, 1)), _full((128, F))],
        out_specs=_row((PH, F)),
        out_shape=jax.ShapeDtypeStruct((Npad, F), jnp.float32),
    )(atoms_p, table)
    s = s0
    v = jnp.zeros((Npad, 3 * F), jnp.float32)

    for l in range(3):
        mp = params['msg'][l]
        up = params['upd'][l]
        phi = pl.pallas_call(
            _phi_kernel,
            grid=(Npad // PH,),
            in_specs=[_row((PH, F)), _full((F, F)), _full((1, F)),
                      _full((F, 2 * F)), _full((1, 2 * F))],
            out_specs=_row((PH, 2 * F)),
            out_shape=jax.ShapeDtypeStruct((Npad, 2 * F), jnp.float32),
        )(s0, mp['sj1']['W'], mp['sj1']['b'].reshape(1, F),
          mp['sj2']['W'][:, F:], mp['sj2']['b'][F:].reshape(1, 2 * F))

        w1r = jnp.pad(jnp.concatenate(
            [mp['rbf']['W'][:, F:2 * F], mp['rbf']['b'][F:2 * F].reshape(1, F)]),
            ((0, 3), (0, 0)))
        w2r = jnp.pad(jnp.concatenate(
            [mp['rbf']['W'][:, 2 * F:], mp['rbf']['b'][2 * F:].reshape(1, F)]),
            ((0, 3), (0, 0)))

        s, v = pl.pallas_call(
            _msg_kernel,
            grid=(NT,),
            in_specs=[
                pl.BlockSpec(memory_space=pltpu.SMEM),
                _row((TR, 4)), _row((TR, 1)), _row((TR, F)), _row((TR, 3 * F)),
                _full((NTC, 4, TC)), _full((NTC, 1, TC)), _full((Npad, 2 * F)),
                _full((24, F)), _full((24, F)),
                _full((F, F)), _full((F, F)), _full((F, F)), _full((F, F)),
                _full((1, F)), _full((F, 3 * F)), _full((1, 3 * F)),
            ],
            out_specs=[_row((TR, F)), _row((TR, 3 * F))],
            out_shape=[jax.ShapeDtypeStruct((Npad, F), jnp.float32),
                       jax.ShapeDtypeStruct((Npad, 3 * F), jnp.float32)],
        )(cinfo, pos_p, gidx_col, s, v, posT3, gidxT3, phi,
          w1r, w2r,
          up['U']['W'], up['V']['W'],
          up['sj1']['W'][:F], up['sj1']['W'][F:], up['sj1']['b'].reshape(1, F),
          up['sj2']['W'], up['sj2']['b'].reshape(1, 3 * F))

    out = pl.pallas_call(
        _head_kernel,
        grid=(Npad // PH,),
        in_specs=[_row((PH, F)), _full((F, F // 2)), _full((1, F // 2)),
                  _full((1, F // 2)), _full((1, 1))],
        out_specs=_row((PH, 1)),
        out_shape=jax.ShapeDtypeStruct((Npad, 1), jnp.float32),
    )(s, params['f1']['W'], params['f1']['b'].reshape(1, F // 2),
      params['f2']['W'].reshape(1, F // 2), params['f2']['b'].reshape(1, 1))

    return out[:N]
